# Initial kernel scaffold; baseline (speedup 1.0000x reference)
import jax
import jax.numpy as jnp
from jax.experimental import pallas as pl


def kernel(local_structs, local_scores, first_window_struct, first_window_scores):
    raise NotImplementedError("write your pallas kernel here")


# speedup vs baseline: 257.8303x; 257.8303x over previous
import jax
import jax.numpy as jnp
from jax.experimental import pallas as pl

_F32 = jnp.float32
_I32 = jnp.int32


def _body(ls_ref, lsc_ref, fws_ref, fwsc_ref, gs_out, gsc_out):
    lsc = lsc_ref[...]
    fws = fws_ref[...]
    fwsc = fwsc_ref[...]
    ls_last = ls_ref[:, :, 50]

    iota_r = jax.lax.broadcasted_iota(_I32, (128, 128), 0)
    iota_c = jax.lax.broadcasted_iota(_I32, (128, 128), 1)

    row_sums = jnp.sum(fwsc, axis=1, keepdims=True)
    lsc0_col = jnp.transpose(lsc[0:1, :])
    gsum0_col = jnp.concatenate([row_sums, row_sums], axis=0) + lsc0_col
    gsum0_row = jnp.transpose(gsum0_col)

    g_i = gsum0_col
    g_j = gsum0_row
    greater = (g_j > g_i) | ((g_j == g_i) & (iota_c < iota_r))
    rank128 = jnp.sum(greater.astype(_I32), axis=1, keepdims=True)
    selected_col = rank128 < 64
    selected_row = jnp.transpose(selected_col)

    fws2 = jnp.concatenate([fws, fws], axis=0)
    fwsc2 = jnp.concatenate([fwsc, fwsc], axis=0)
    lsl0_col = jnp.transpose(ls_last[0:1, :])
    data = jnp.concatenate(
        [fws2, lsl0_col, fwsc2, lsc0_col, gsum0_col], axis=1)
    data_t = jnp.transpose(data)

    r = jnp.zeros((128, 128), _F32)
    nd = jnp.ones((128, 128), _F32)
    for c in range(105):
        a_c = data[:, c:c + 1]
        b_c = data_t[c:c + 1, :]
        r = r + nd * jnp.sign(a_c - b_c)
        nd = nd * (a_c == b_c).astype(_F32)
    less = r < 0

    rank_sel = jnp.sum((less & selected_col).astype(_I32), axis=0,
                       keepdims=True)
    iota64_r = jax.lax.broadcasted_iota(_I32, (64, 128), 0)
    p_mat = ((rank_sel == iota64_r) & selected_row).astype(_F32)
    gs_sorted = jax.lax.dot(p_mat, data[:, 0:52],
                            precision=jax.lax.Precision.HIGHEST)
    gsc_sorted = jax.lax.dot(p_mat, data[:, 52:104],
                             precision=jax.lax.Precision.HIGHEST)
    gsum_sorted = jax.lax.dot(p_mat, data[:, 104:105],
                              precision=jax.lax.Precision.HIGHEST)

    gsum_sorted_row = jnp.transpose(gsum_sorted)
    lsc1_2x64 = jnp.concatenate([lsc[1:2, 0:64], lsc[1:2, 64:128]], axis=0)
    v1 = gsum_sorted_row + lsc1_2x64
    idxj = (jax.lax.broadcasted_iota(_I32, (2, 64), 0) * 64
            + jax.lax.broadcasted_iota(_I32, (2, 64), 1))
    v1_max = jnp.max(v1)
    c1 = jnp.min(jnp.where(v1 == v1_max, idxj, 999))
    m1 = c1 % 64

    iota64_c1 = jax.lax.broadcasted_iota(_I32, (64, 1), 0)
    e_col = (iota64_c1 == m1).astype(_F32)
    prefix_gs = jnp.sum(e_col * gs_sorted, axis=0, keepdims=True)
    prefix_gsc = jnp.sum(e_col * gsc_sorted, axis=0, keepdims=True)

    iota128_row = jax.lax.broadcasted_iota(_I32, (1, 128), 1)
    sel_c1 = (iota128_row == c1).astype(_F32)
    lsl1_c1 = jnp.sum(sel_c1 * ls_last[1:2, :], axis=1, keepdims=True)
    lsc1_c1 = jnp.sum(sel_c1 * lsc[1:2, :], axis=1, keepdims=True)

    gmax = jnp.max(lsc, axis=1, keepdims=True)
    cidx = jnp.min(jnp.where(lsc == gmax, iota_c, 999), axis=1,
                   keepdims=True)
    onehot = (iota_c == cidx).astype(_F32)
    picks_struct_col = jnp.sum(onehot * ls_last, axis=1, keepdims=True)
    picks_struct_row = jnp.transpose(picks_struct_col)
    picks_score_row = jnp.transpose(gmax)

    row_gs = jnp.concatenate(
        [prefix_gs, lsl1_c1, picks_struct_row[:, 2:128]], axis=1)
    row_gsc = jnp.concatenate(
        [prefix_gsc, lsc1_c1, picks_score_row[:, 2:128]], axis=1)
    gs_out[...] = jnp.broadcast_to(row_gs, (64, 179))
    gsc_out[...] = jnp.broadcast_to(row_gsc, (64, 179))


def kernel(local_structs, local_scores, first_window_struct, first_window_scores):
    out_shape = (
        jax.ShapeDtypeStruct((64, 179), _F32),
        jax.ShapeDtypeStruct((64, 179), _F32),
    )
    return pl.pallas_call(_body, out_shape=out_shape)(
        local_structs, local_scores, first_window_struct, first_window_scores)


# speedup vs baseline: 285.9587x; 1.1091x over previous
import jax
import jax.numpy as jnp
from jax.experimental import pallas as pl

_F32 = jnp.float32
_I32 = jnp.int32


def _body(ls_ref, lsc_ref, fws_ref, fwsc_ref, gs_out, gsc_out):
    lsc = lsc_ref[...]
    fws = fws_ref[...]
    fwsc = fwsc_ref[...]
    ls_last = ls_ref[:, :, 50]

    iota_r = jax.lax.broadcasted_iota(_I32, (128, 128), 0)
    iota_c = jax.lax.broadcasted_iota(_I32, (128, 128), 1)

    row_sums = jnp.sum(fwsc, axis=1, keepdims=True)
    lsc0_col = jnp.transpose(lsc[0:1, :])
    gsum0_col = jnp.concatenate([row_sums, row_sums], axis=0) + lsc0_col
    gsum0_row = jnp.transpose(gsum0_col)

    g_i = gsum0_col
    g_j = gsum0_row
    greater = (g_j > g_i) | ((g_j == g_i) & (iota_c < iota_r))
    rank128 = jnp.sum(greater.astype(_I32), axis=1, keepdims=True)
    selected_col = rank128 < 64
    selected_row = jnp.transpose(selected_col)

    fws2 = jnp.concatenate([fws, fws], axis=0)
    fwsc2 = jnp.concatenate([fwsc, fwsc], axis=0)
    lsl0_col = jnp.transpose(ls_last[0:1, :])
    data = jnp.concatenate(
        [fws2, lsl0_col, fwsc2, lsc0_col, gsum0_col], axis=1)

    fws_t = jnp.transpose(fws)
    r64 = jnp.zeros((64, 64), _F32)
    nd64 = jnp.ones((64, 64), _F32)
    for c in range(51):
        a_c = fws[:, c:c + 1]
        b_c = fws_t[c:c + 1, :]
        r64 = r64 + nd64 * jnp.sign(a_c - b_c)
        nd64 = nd64 * (a_c == b_c).astype(_F32)
    less64 = (r64 < 0).astype(_F32)
    less64_2 = jnp.concatenate([less64, less64], axis=0)
    less64_4 = jnp.concatenate([less64_2, less64_2], axis=1)
    lessd = (lsl0_col < jnp.transpose(lsl0_col)).astype(_F32)
    same_m = ((iota_r % 64) == (iota_c % 64)).astype(_F32)
    less = same_m * lessd + (1.0 - same_m) * less64_4

    rank_sel = jnp.sum(less * selected_col.astype(_F32), axis=0,
                       keepdims=True).astype(_I32)
    iota64_r = jax.lax.broadcasted_iota(_I32, (64, 128), 0)
    p_mat = ((rank_sel == iota64_r) & selected_row).astype(_F32)
    gs_sorted = jax.lax.dot(p_mat, data[:, 0:52],
                            precision=jax.lax.Precision.HIGHEST)
    gsc_sorted = jax.lax.dot(p_mat, data[:, 52:104],
                             precision=jax.lax.Precision.HIGHEST)
    gsum_sorted = jax.lax.dot(p_mat, data[:, 104:105],
                              precision=jax.lax.Precision.HIGHEST)

    gsum_sorted_row = jnp.transpose(gsum_sorted)
    lsc1_2x64 = jnp.concatenate([lsc[1:2, 0:64], lsc[1:2, 64:128]], axis=0)
    v1 = gsum_sorted_row + lsc1_2x64
    idxj = (jax.lax.broadcasted_iota(_I32, (2, 64), 0) * 64
            + jax.lax.broadcasted_iota(_I32, (2, 64), 1))
    v1_max = jnp.max(v1)
    c1 = jnp.min(jnp.where(v1 == v1_max, idxj, 999))
    m1 = c1 % 64

    iota64_c1 = jax.lax.broadcasted_iota(_I32, (64, 1), 0)
    e_col = (iota64_c1 == m1).astype(_F32)
    prefix_gs = jnp.sum(e_col * gs_sorted, axis=0, keepdims=True)
    prefix_gsc = jnp.sum(e_col * gsc_sorted, axis=0, keepdims=True)

    iota128_row = jax.lax.broadcasted_iota(_I32, (1, 128), 1)
    sel_c1 = (iota128_row == c1).astype(_F32)
    lsl1_c1 = jnp.sum(sel_c1 * ls_last[1:2, :], axis=1, keepdims=True)
    lsc1_c1 = jnp.sum(sel_c1 * lsc[1:2, :], axis=1, keepdims=True)

    gmax = jnp.max(lsc, axis=1, keepdims=True)
    cidx = jnp.min(jnp.where(lsc == gmax, iota_c, 999), axis=1,
                   keepdims=True)
    onehot = (iota_c == cidx).astype(_F32)
    picks_struct_col = jnp.sum(onehot * ls_last, axis=1, keepdims=True)
    picks_struct_row = jnp.transpose(picks_struct_col)
    picks_score_row = jnp.transpose(gmax)

    row_gs = jnp.concatenate(
        [prefix_gs, lsl1_c1, picks_struct_row[:, 2:128]], axis=1)
    row_gsc = jnp.concatenate(
        [prefix_gsc, lsc1_c1, picks_score_row[:, 2:128]], axis=1)
    gs_out[...] = jnp.broadcast_to(row_gs, (64, 179))
    gsc_out[...] = jnp.broadcast_to(row_gsc, (64, 179))


def kernel(local_structs, local_scores, first_window_struct, first_window_scores):
    out_shape = (
        jax.ShapeDtypeStruct((64, 179), _F32),
        jax.ShapeDtypeStruct((64, 179), _F32),
    )
    return pl.pallas_call(_body, out_shape=out_shape)(
        local_structs, local_scores, first_window_struct, first_window_scores)
